# baseline (speedup 1.0000x reference)
import functools

import jax
import jax.numpy as jnp
from jax import lax
from jax.experimental import pallas as pl
from jax.experimental.pallas import tpu as pltpu
from jax.experimental.pallas import tpu_sc as plsc

F32 = jnp.float32

N = 10000
E = 320000
F = 128
K = 8000
NC = 2
NS = 16
NW = NC * NS
EPW = E // NW
B = 80
ITERS = EPW // B
N_PAD = 10240
RPT = N_PAD // NS
_ZCP = [(j * B, B) for j in range(RPT // B)]
if RPT % B:
    _ZCP.append(((RPT // B) * B, RPT % B))


def _mesh():
    return plsc.VectorSubcoreMesh(
        core_axis_name="c", subcore_axis_name="s", num_cores=NC,
        num_subcores=NS)


_OFFS = list(range(0, B - 15, 16)) + ([B - 16] if B % 16 else [])


def _make_sc_scatter(feat, depth, gdist):
    if feat is None:
        out_shape = (2, N_PAD)
        agg_t = pltpu.VMEM_SHARED((N_PAD,), F32)
        rows_t = pltpu.VMEM((B,), F32)
    else:
        out_shape = (2, N_PAD, feat)
        agg_t = pltpu.VMEM_SHARED((N_PAD, feat), F32)
        rows_t = pltpu.VMEM((B, feat), F32)
    D, G = depth, gdist
    nzc = len(_ZCP)

    def body(y_hbm, pk_hbm, out_hbm, agg_sh, pidx, *rest):
        sidx = rest[0:D]
        didx = rest[D:2 * D]
        rows = rest[2 * D:3 * D]
        gsem = rest[3 * D:4 * D]
        ssem = rest[4 * D:5 * D]
        cid = lax.axis_index("c")
        sid = lax.axis_index("s")
        wid = sid * NC + cid

        pltpu.sync_copy(
            pk_hbm.at[pl.ds(pl.multiple_of(wid * EPW, 8), EPW)], pidx)

        zv = jnp.zeros((16,), F32)
        for s in range(D):
            if feat is None:
                for o in _OFFS:
                    rows[s][pl.ds(o, 16)] = zv
            else:
                def zb(i, c, _s=s):
                    for o in range(feat // 16):
                        rows[_s][i, pl.ds(o * 16, 16)] = zv
                    return c
                lax.fori_loop(0, B, zb, 0)

        def z_args(j):
            s = j % D
            off, sz = _ZCP[j]
            if sz == B:
                srcr = rows[s]
            elif feat is None:
                srcr = rows[s].at[pl.ds(0, sz)]
            else:
                srcr = rows[s].at[pl.ds(0, sz), :]
            return (srcr,
                    agg_sh.at[pl.ds(pl.multiple_of(sid * RPT + off, 8), sz)],
                    gsem[s])
        for j in range(nzc):
            if j >= D:
                pltpu.make_async_copy(*z_args(j - D)).wait()
            pltpu.async_copy(*z_args(j))
        for j in range(max(0, nzc - D), nzc):
            pltpu.make_async_copy(*z_args(j)).wait()
        plsc.subcore_barrier()

        mask16 = jnp.int32(0xFFFF)

        def unpack(it, buf):
            for o in _OFFS:
                w = pidx[pl.ds(pl.multiple_of(it * B + o, 8), 16)]
                sidx[buf][pl.ds(o, 16)] = w & mask16
                didx[buf][pl.ds(o, 16)] = lax.shift_right_logical(w, 16)

        def g_desc(buf):
            return pltpu.make_async_copy(
                y_hbm.at[sidx[buf]], rows[buf], gsem[buf])

        def s_desc(buf):
            return pltpu.make_async_copy(
                rows[buf], agg_sh.at[didx[buf]], ssem[buf])

        def start_g(buf):
            pltpu.async_copy(y_hbm.at[sidx[buf]], rows[buf], gsem[buf])

        def start_s(buf):
            pltpu.async_copy(rows[buf], agg_sh.at[didx[buf]], ssem[buf],
                             add=True)

        for it in range(D):
            buf = it % D
            unpack(it, buf)
            start_g(buf)
            if it >= G:
                jbuf = (buf - G) % D
                g_desc(jbuf).wait()
                start_s(jbuf)

        n_steady = (ITERS - D) // D

        def steady(k, c):
            for off in range(D):
                it = D + D * k + off
                buf = off
                jbuf = (off - G) % D
                s_desc(buf).wait()
                unpack(it, buf)
                start_g(buf)
                g_desc(jbuf).wait()
                start_s(jbuf)
            return c
        lax.fori_loop(0, n_steady, steady, 0)

        for it in range(D + n_steady * D, ITERS):
            buf = it % D
            jbuf = (buf - G) % D
            s_desc(buf).wait()
            unpack(it, buf)
            start_g(buf)
            g_desc(jbuf).wait()
            start_s(jbuf)
        for j in range(ITERS - G, ITERS):
            buf = j % D
            g_desc(buf).wait()
            start_s(buf)
        for j in range(ITERS - D, ITERS):
            s_desc(j % D).wait()

        plsc.subcore_barrier()
        tb = pl.multiple_of(sid * RPT, 8)
        pltpu.sync_copy(agg_sh.at[pl.ds(tb, RPT)],
                        out_hbm.at[cid, pl.ds(tb, RPT)])

    scratch = [agg_t, pltpu.VMEM((EPW,), jnp.int32)]
    scratch += [pltpu.VMEM((B,), jnp.int32) for _ in range(2 * D)]
    scratch += [rows_t for _ in range(D)]
    scratch += [pltpu.SemaphoreType.DMA for _ in range(2 * D)]

    return pl.kernel(
        body,
        out_type=jax.ShapeDtypeStruct(out_shape, F32),
        mesh=_mesh(),
        scratch_types=scratch,
    )


def _make_sc_deg(D):
    def body(y_hbm, pk_hbm, out_hbm, agg_sh, pidx, ystage, *rest):
        didx = rest[0:D]
        vals = rest[D:2 * D]
        ssem = rest[2 * D:3 * D]
        cid = lax.axis_index("c")
        sid = lax.axis_index("s")
        wid = sid * NC + cid

        pltpu.sync_copy(
            pk_hbm.at[pl.ds(pl.multiple_of(wid * EPW, 8), EPW)], pidx)
        pltpu.sync_copy(y_hbm, ystage)

        zv = jnp.zeros((16,), F32)
        for s in range(D):
            for o in _OFFS:
                vals[s][pl.ds(o, 16)] = zv

        def z_args(j):
            s = j % D
            off, sz = _ZCP[j]
            srcr = vals[s] if sz == B else vals[s].at[pl.ds(0, sz)]
            return (srcr,
                    agg_sh.at[pl.ds(pl.multiple_of(sid * RPT + off, 8), sz)],
                    ssem[s])
        nzc = len(_ZCP)
        for j in range(nzc):
            if j >= D:
                pltpu.make_async_copy(*z_args(j - D)).wait()
            pltpu.async_copy(*z_args(j))
        for j in range(max(0, nzc - D), nzc):
            pltpu.make_async_copy(*z_args(j)).wait()
        plsc.subcore_barrier()

        mask16 = jnp.int32(0xFFFF)

        def s_desc(buf):
            return pltpu.make_async_copy(
                vals[buf], agg_sh.at[didx[buf]], ssem[buf])

        def step(it, buf):
            for o in _OFFS:
                w = pidx[pl.ds(pl.multiple_of(it * B + o, 8), 16)]
                vals[buf][pl.ds(o, 16)] = plsc.load_gather(
                    ystage, [w & mask16])
                didx[buf][pl.ds(o, 16)] = lax.shift_right_logical(w, 16)
            pltpu.async_copy(vals[buf], agg_sh.at[didx[buf]], ssem[buf],
                             add=True)

        for it in range(D):
            step(it, it % D)

        n_steady = (ITERS - D) // D

        def steady(k, c):
            for off in range(D):
                it = D + D * k + off
                s_desc(off).wait()
                step(it, off)
            return c
        lax.fori_loop(0, n_steady, steady, 0)

        for it in range(D + n_steady * D, ITERS):
            s_desc(it % D).wait()
            step(it, it % D)
        for j in range(ITERS - D, ITERS):
            s_desc(j % D).wait()

        plsc.subcore_barrier()
        tb = pl.multiple_of(sid * RPT, 8)
        pltpu.sync_copy(agg_sh.at[pl.ds(tb, RPT)],
                        out_hbm.at[cid, pl.ds(tb, RPT)])

    scratch = [pltpu.VMEM_SHARED((N_PAD,), F32),
               pltpu.VMEM((EPW,), jnp.int32),
               pltpu.VMEM((N,), F32)]
    scratch += [pltpu.VMEM((B,), jnp.int32) for _ in range(D)]
    scratch += [pltpu.VMEM((B,), F32) for _ in range(D)]
    scratch += [pltpu.SemaphoreType.DMA for _ in range(D)]

    return pl.kernel(
        body,
        out_type=jax.ShapeDtypeStruct((2, N_PAD), F32),
        mesh=_mesh(),
        scratch_types=scratch,
        compiler_params=pltpu.CompilerParams(needs_layout_passes=False),
    )


@functools.cache
def _get_sc_scatter(feat):
    if feat is None:
        return _make_sc_deg(8)
    return _make_sc_scatter(feat, depth=3, gdist=2)


NB = 10
BN = N // NB


def _tc_call(f, out_shapes):
    return pl.pallas_call(f, out_shape=out_shapes)


def _vspec():
    return pl.BlockSpec((BN, 1), lambda i: (i, 0))


def _fspec():
    return pl.BlockSpec((BN, F), lambda i: (i, 0))


def _pspec(w):
    return pl.BlockSpec((2, BN, w), lambda i: (0, i, 0))


def _wspec(shape):
    return pl.BlockSpec(shape, lambda i: tuple(0 for _ in shape))


def _tca_body(x_ref, w1_ref, degp_ref, y1_ref, dinv0_ref):
    deg = degp_ref[0] + degp_ref[1] + 1.0
    dinv = lax.rsqrt(deg)
    xw = jnp.dot(x_ref[...], w1_ref[...], preferred_element_type=F32)
    y1_ref[...] = dinv * xw
    dinv0_ref[...] = dinv


def _tcb_body(aggp_ref, y1_ref, dinv0_ref, b1_ref, p_ref, w2_ref, xw2_ref,
              m_ref):
    agg = aggp_ref[0, :N, :] + aggp_ref[1, :N, :]
    dinv = dinv0_ref[...]
    h = jnp.maximum(dinv * (agg + y1_ref[...]) + b1_ref[...][None, :], 0.0)
    p = p_ref[...]
    pnorm = jnp.sqrt(jnp.sum(p * p))
    z = jnp.dot(h, p[:, None], preferred_element_type=F32) / pnorm
    score = jnp.tanh(z)
    u = lax.bitcast_convert_type(z, jnp.uint32)
    top = jnp.uint32(0x80000000)
    k_key = jnp.where(u >= top, ~u, u | top)

    def bis(i, t):
        cand = t | (jnp.uint32(1) << (31 - i).astype(jnp.uint32))
        cnt = jnp.sum((k_key >= cand).astype(jnp.int32))
        return jnp.where(cnt >= K, cand, t)
    t_thr = lax.fori_loop(0, 32, bis, jnp.uint32(0))

    gt = k_key > t_thr
    eqm = k_key == t_thr
    need = K - jnp.sum(gt.astype(jnp.int32))
    idx = lax.broadcasted_iota(jnp.int32, (N, 1), 0)

    def bis2(i, xv):
        cand = xv | (jnp.int32(1) << (13 - i))
        cnt = jnp.sum((eqm & (idx < cand)).astype(jnp.int32))
        return jnp.where(cnt <= need, cand, xv)
    x_thr = lax.fori_loop(0, 14, bis2, jnp.int32(0))

    m = (gt | (eqm & (idx < x_thr))).astype(F32)
    m_ref[...] = m
    hp = h * (score * m)
    xw2_ref[...] = jnp.dot(hp, w2_ref[...], preferred_element_type=F32)


def _tcd_body(degap_ref, m_ref, xw2_ref, y2_ref, dinvp_ref):
    m = m_ref[...]
    da = degap_ref[0] + degap_ref[1]
    dinv = lax.rsqrt(m * da + 1.0)
    dinvp_ref[...] = dinv
    y2_ref[...] = (dinv * m) * xw2_ref[...]


def _tce_body(aggp_ref, yp_ref, dinvp_ref, m_ref, b_ref, wn_ref, yn_ref):
    agg = aggp_ref[0] + aggp_ref[1]
    dinv = dinvp_ref[...]
    m = m_ref[...]
    h = jnp.maximum(dinv * (m * agg + yp_ref[...]) + b_ref[...][None, :], 0.0)
    yn_ref[...] = (dinv * m) * jnp.dot(h, wn_ref[...],
                                       preferred_element_type=F32)


def _tcg_body(aggp_ref, y4_ref, dinvp_ref, m_ref, b4_ref, out_ref):
    agg = aggp_ref[0] + aggp_ref[1]
    dinv = dinvp_ref[...]
    m = m_ref[...]
    h4 = dinv * (m * agg + y4_ref[...]) + b4_ref[...][None, :]
    s = lax.dot_general(m, h4, (((0,), (0,)), ((), ())),
                        preferred_element_type=F32)
    @pl.when(pl.program_id(0) == 0)
    def _():
        out_ref[...] = jnp.zeros_like(out_ref)
    out_ref[...] += s / F32(K)


def _build_tc(interpret=False):
    tca = pl.pallas_call(
        _tca_body,
        grid=(NB,),
        in_specs=[_fspec(), _wspec((F, F)), _pspec(1)],
        out_specs=(_fspec(), _vspec()),
        out_shape=(jax.ShapeDtypeStruct((N, F), F32),
                   jax.ShapeDtypeStruct((N, 1), F32)),
        interpret=interpret)
    tcb = pl.pallas_call(
        _tcb_body,
        out_shape=(jax.ShapeDtypeStruct((N, F), F32),
                   jax.ShapeDtypeStruct((N, 1), F32)),
        interpret=interpret)
    tcd = pl.pallas_call(
        _tcd_body,
        grid=(NB,),
        in_specs=[_pspec(1), _vspec(), _fspec()],
        out_specs=(_fspec(), _vspec()),
        out_shape=(jax.ShapeDtypeStruct((N, F), F32),
                   jax.ShapeDtypeStruct((N, 1), F32)),
        interpret=interpret)
    tce = pl.pallas_call(
        _tce_body,
        grid=(NB,),
        in_specs=[_pspec(F), _fspec(), _vspec(), _vspec(), _wspec((F,)),
                  _wspec((F, F))],
        out_specs=_fspec(),
        out_shape=jax.ShapeDtypeStruct((N, F), F32),
        interpret=interpret)
    tcg = pl.pallas_call(
        _tcg_body,
        grid=(NB,),
        in_specs=[_pspec(F), _fspec(), _vspec(), _vspec(), _wspec((F,))],
        out_specs=pl.BlockSpec((1, F), lambda i: (0, 0)),
        out_shape=jax.ShapeDtypeStruct((1, F), F32),
        interpret=interpret)
    return tca, tcb, tcd, tce, tcg


_tca, _tcb, _tcd, _tce, _tcg = _build_tc()


def kernel(x, edge_index, batch, W1, b1, p, W2, b2, W3, b3, W4, b4):
    del batch
    _sc_rows = _get_sc_scatter(F)
    _sc_deg = _get_sc_scatter(None)
    src = edge_index[0]
    dst = edge_index[1]
    packed = src | (dst << 16)
    ones_n = jnp.ones((N,), F32)

    deg0p = _sc_deg(ones_n, packed)
    y1, dinv0 = _tca(x, W1, deg0p.reshape(2, N_PAD, 1))
    agg1p = _sc_rows(y1, packed)
    xw2, m = _tcb(agg1p, y1, dinv0, b1, p, W2)
    degap = _sc_deg(m.reshape(N), packed)
    y2, dinvp = _tcd(degap.reshape(2, N_PAD, 1), m, xw2)
    agg2p = _sc_rows(y2, packed)
    y3 = _tce(agg2p, y2, dinvp, m, b2, W3)
    agg3p = _sc_rows(y3, packed)
    y4 = _tce(agg3p, y3, dinvp, m, b3, W4)
    agg4p = _sc_rows(y4, packed)
    return _tcg(agg4p, y4, dinvp, m, b4)


# speedup vs baseline: 1.0934x; 1.0934x over previous
import functools

import jax
import jax.numpy as jnp
from jax import lax
from jax.experimental import pallas as pl
from jax.experimental.pallas import tpu as pltpu
from jax.experimental.pallas import tpu_sc as plsc

F32 = jnp.float32

N = 10000
E = 320000
F = 128
K = 8000
NC = 2
NS = 16
NW = NC * NS
EPW = E // NW
B = 80
ITERS = EPW // B
N_PAD = 10240
RPT = N_PAD // NS
_ZCP = [(j * B, B) for j in range(RPT // B)]
if RPT % B:
    _ZCP.append(((RPT // B) * B, RPT % B))


def _mesh():
    return plsc.VectorSubcoreMesh(
        core_axis_name="c", subcore_axis_name="s", num_cores=NC,
        num_subcores=NS)


_OFFS = list(range(0, B - 15, 16)) + ([B - 16] if B % 16 else [])


def _make_sc_scatter(feat, depth, gdist):
    if feat is None:
        out_shape = (2, N_PAD)
        agg_t = pltpu.VMEM_SHARED((N_PAD,), F32)
        rows_t = pltpu.VMEM((B,), F32)
    else:
        out_shape = (2, N_PAD, feat)
        agg_t = pltpu.VMEM_SHARED((N_PAD, feat), F32)
        rows_t = pltpu.VMEM((B, feat), F32)
    D, G = depth, gdist
    nzc = len(_ZCP)

    def body(y_hbm, pk_hbm, out_hbm, agg_sh, pidx, *rest):
        sidx = rest[0:D]
        didx = rest[D:2 * D]
        rows = rest[2 * D:3 * D]
        gsem = rest[3 * D:4 * D]
        ssem = rest[4 * D:5 * D]
        cid = lax.axis_index("c")
        sid = lax.axis_index("s")
        wid = sid * NC + cid

        pltpu.sync_copy(
            pk_hbm.at[pl.ds(pl.multiple_of(wid * EPW, 8), EPW)], pidx)

        zv = jnp.zeros((16,), F32)
        for s in range(D):
            if feat is None:
                for o in _OFFS:
                    rows[s][pl.ds(o, 16)] = zv
            else:
                def zb(i, c, _s=s):
                    for o in range(feat // 16):
                        rows[_s][i, pl.ds(o * 16, 16)] = zv
                    return c
                lax.fori_loop(0, B, zb, 0)

        def z_args(j):
            s = j % D
            off, sz = _ZCP[j]
            if sz == B:
                srcr = rows[s]
            elif feat is None:
                srcr = rows[s].at[pl.ds(0, sz)]
            else:
                srcr = rows[s].at[pl.ds(0, sz), :]
            return (srcr,
                    agg_sh.at[pl.ds(pl.multiple_of(sid * RPT + off, 8), sz)],
                    gsem[s])
        for j in range(nzc):
            if j >= D:
                pltpu.make_async_copy(*z_args(j - D)).wait()
            pltpu.async_copy(*z_args(j))
        for j in range(max(0, nzc - D), nzc):
            pltpu.make_async_copy(*z_args(j)).wait()
        plsc.subcore_barrier()

        mask16 = jnp.int32(0xFFFF)

        def unpack(it, buf):
            for o in _OFFS:
                w = pidx[pl.ds(pl.multiple_of(it * B + o, 8), 16)]
                sidx[buf][pl.ds(o, 16)] = w & mask16
                didx[buf][pl.ds(o, 16)] = lax.shift_right_logical(w, 16)

        def g_desc(buf):
            return pltpu.make_async_copy(
                y_hbm.at[sidx[buf]], rows[buf], gsem[buf])

        def s_desc(buf):
            return pltpu.make_async_copy(
                rows[buf], agg_sh.at[didx[buf]], ssem[buf])

        def start_g(buf):
            pltpu.async_copy(y_hbm.at[sidx[buf]], rows[buf], gsem[buf])

        def start_s(buf):
            pltpu.async_copy(rows[buf], agg_sh.at[didx[buf]], ssem[buf],
                             add=True)

        for it in range(D):
            buf = it % D
            unpack(it, buf)
            start_g(buf)
            if it >= G:
                jbuf = (buf - G) % D
                g_desc(jbuf).wait()
                start_s(jbuf)

        n_steady = (ITERS - D) // D

        def steady(k, c):
            for off in range(D):
                it = D + D * k + off
                buf = off
                jbuf = (off - G) % D
                s_desc(buf).wait()
                unpack(it, buf)
                start_g(buf)
                g_desc(jbuf).wait()
                start_s(jbuf)
            return c
        lax.fori_loop(0, n_steady, steady, 0)

        for it in range(D + n_steady * D, ITERS):
            buf = it % D
            jbuf = (buf - G) % D
            s_desc(buf).wait()
            unpack(it, buf)
            start_g(buf)
            g_desc(jbuf).wait()
            start_s(jbuf)
        for j in range(ITERS - G, ITERS):
            buf = j % D
            g_desc(buf).wait()
            start_s(buf)
        for j in range(ITERS - D, ITERS):
            s_desc(j % D).wait()

        plsc.subcore_barrier()
        tb = pl.multiple_of(sid * RPT, 8)
        pltpu.sync_copy(agg_sh.at[pl.ds(tb, RPT)],
                        out_hbm.at[cid, pl.ds(tb, RPT)])

    scratch = [agg_t, pltpu.VMEM((EPW,), jnp.int32)]
    scratch += [pltpu.VMEM((B,), jnp.int32) for _ in range(2 * D)]
    scratch += [rows_t for _ in range(D)]
    scratch += [pltpu.SemaphoreType.DMA for _ in range(2 * D)]

    return pl.kernel(
        body,
        out_type=jax.ShapeDtypeStruct(out_shape, F32),
        mesh=_mesh(),
        scratch_types=scratch,
    )


def _make_sc_deg(D):
    def body(y_hbm, pk_hbm, out_hbm, agg_sh, pidx, ystage, *rest):
        didx = rest[0:D]
        vals = rest[D:2 * D]
        ssem = rest[2 * D:3 * D]
        cid = lax.axis_index("c")
        sid = lax.axis_index("s")
        wid = sid * NC + cid

        pltpu.sync_copy(
            pk_hbm.at[pl.ds(pl.multiple_of(wid * EPW, 8), EPW)], pidx)
        pltpu.sync_copy(y_hbm, ystage)

        zv = jnp.zeros((16,), F32)
        for s in range(D):
            for o in _OFFS:
                vals[s][pl.ds(o, 16)] = zv

        def z_args(j):
            s = j % D
            off, sz = _ZCP[j]
            srcr = vals[s] if sz == B else vals[s].at[pl.ds(0, sz)]
            return (srcr,
                    agg_sh.at[pl.ds(pl.multiple_of(sid * RPT + off, 8), sz)],
                    ssem[s])
        nzc = len(_ZCP)
        for j in range(nzc):
            if j >= D:
                pltpu.make_async_copy(*z_args(j - D)).wait()
            pltpu.async_copy(*z_args(j))
        for j in range(max(0, nzc - D), nzc):
            pltpu.make_async_copy(*z_args(j)).wait()
        plsc.subcore_barrier()

        mask16 = jnp.int32(0xFFFF)

        def s_desc(buf):
            return pltpu.make_async_copy(
                vals[buf], agg_sh.at[didx[buf]], ssem[buf])

        def step(it, buf):
            for o in _OFFS:
                w = pidx[pl.ds(pl.multiple_of(it * B + o, 8), 16)]
                vals[buf][pl.ds(o, 16)] = plsc.load_gather(
                    ystage, [w & mask16])
                didx[buf][pl.ds(o, 16)] = lax.shift_right_logical(w, 16)
            pltpu.async_copy(vals[buf], agg_sh.at[didx[buf]], ssem[buf],
                             add=True)

        for it in range(D):
            step(it, it % D)

        n_steady = (ITERS - D) // D

        def steady(k, c):
            for off in range(D):
                it = D + D * k + off
                s_desc(off).wait()
                step(it, off)
            return c
        lax.fori_loop(0, n_steady, steady, 0)

        for it in range(D + n_steady * D, ITERS):
            s_desc(it % D).wait()
            step(it, it % D)
        for j in range(ITERS - D, ITERS):
            s_desc(j % D).wait()

        plsc.subcore_barrier()
        tb = pl.multiple_of(sid * RPT, 8)
        pltpu.sync_copy(agg_sh.at[pl.ds(tb, RPT)],
                        out_hbm.at[cid, pl.ds(tb, RPT)])

    scratch = [pltpu.VMEM_SHARED((N_PAD,), F32),
               pltpu.VMEM((EPW,), jnp.int32),
               pltpu.VMEM((N,), F32)]
    scratch += [pltpu.VMEM((B,), jnp.int32) for _ in range(D)]
    scratch += [pltpu.VMEM((B,), F32) for _ in range(D)]
    scratch += [pltpu.SemaphoreType.DMA for _ in range(D)]

    return pl.kernel(
        body,
        out_type=jax.ShapeDtypeStruct((2, N_PAD), F32),
        mesh=_mesh(),
        scratch_types=scratch,
        compiler_params=pltpu.CompilerParams(needs_layout_passes=False),
    )


@functools.cache
def _get_sc_scatter(feat):
    if feat is None:
        return _make_sc_deg(8)
    return _make_sc_scatter(feat, depth=3, gdist=2)


NB = 10
BN = N // NB


def _tc_call(f, out_shapes):
    return pl.pallas_call(f, out_shape=out_shapes)


def _vspec():
    return pl.BlockSpec((BN, 1), lambda i: (i, 0))


def _fspec():
    return pl.BlockSpec((BN, F), lambda i: (i, 0))


def _pspec(w):
    return pl.BlockSpec((2, BN, w), lambda i: (0, i, 0))


def _wspec(shape):
    return pl.BlockSpec(shape, lambda i: tuple(0 for _ in shape))


def _tca_body(x_ref, w1_ref, degp_ref, y1_ref, dinv0_ref):
    deg = degp_ref[0] + degp_ref[1] + 1.0
    dinv = lax.rsqrt(deg)
    xw = jnp.dot(x_ref[...], w1_ref[...], preferred_element_type=F32)
    y1_ref[...] = dinv * xw
    dinv0_ref[...] = dinv


def _tcb_body(aggp_ref, y1_ref, dinv0_ref, b1_ref, p_ref, h_ref, sm_ref):
    agg = aggp_ref[0, :N, :] + aggp_ref[1, :N, :]
    dinv = dinv0_ref[...]
    h = jnp.maximum(dinv * (agg + y1_ref[...]) + b1_ref[...][None, :], 0.0)
    h_ref[...] = h
    p = p_ref[...]
    pnorm = jnp.sqrt(jnp.sum(p * p))
    z = lax.dot_general(p[None, :], h, (((1,), (1,)), ((), ())),
                        preferred_element_type=F32) / pnorm
    score = jnp.tanh(z)
    u = lax.bitcast_convert_type(z, jnp.uint32)
    top = jnp.uint32(0x80000000)
    k_key = jnp.where(u >= top, ~u, u | top)

    def bis(i, t):
        cand = t | (jnp.uint32(1) << (31 - i).astype(jnp.uint32))
        cnt = jnp.sum((k_key >= cand).astype(jnp.int32))
        return jnp.where(cnt >= K, cand, t)
    t_thr = lax.fori_loop(0, 32, bis, jnp.uint32(0))

    gt = k_key > t_thr
    eqm = k_key == t_thr
    need = K - jnp.sum(gt.astype(jnp.int32))
    idx = lax.broadcasted_iota(jnp.int32, (1, N), 1)

    def bis2(i, xv):
        cand = xv | (jnp.int32(1) << (13 - i))
        cnt = jnp.sum((eqm & (idx < cand)).astype(jnp.int32))
        return jnp.where(cnt <= need, cand, xv)
    x_thr = lax.fori_loop(0, 14, bis2, jnp.int32(0))

    m = (gt | (eqm & (idx < x_thr))).astype(F32)
    sm_ref[...] = jnp.concatenate([m, score * m], axis=0)


def _tcc_body(h_ref, sm_ref, w2_ref, xw2_ref):
    xw2_ref[...] = jnp.dot(h_ref[...] * sm_ref[...],
                           w2_ref[...], preferred_element_type=F32)


def _tcp_body(ei_ref, pk_ref):
    pk_ref[...] = ei_ref[0] | (ei_ref[1] << 16)


def _tcd_body(degap_ref, m_ref, xw2_ref, y2_ref, dinvp_ref):
    m = m_ref[...]
    da = degap_ref[0] + degap_ref[1]
    dinv = lax.rsqrt(m * da + 1.0)
    dinvp_ref[...] = dinv
    y2_ref[...] = (dinv * m) * xw2_ref[...]


def _tce_body(aggp_ref, yp_ref, dinvp_ref, m_ref, b_ref, wn_ref, yn_ref):
    agg = aggp_ref[0] + aggp_ref[1]
    dinv = dinvp_ref[...]
    m = m_ref[...]
    h = jnp.maximum(dinv * (m * agg + yp_ref[...]) + b_ref[...][None, :], 0.0)
    yn_ref[...] = (dinv * m) * jnp.dot(h, wn_ref[...],
                                       preferred_element_type=F32)


def _tcg_body(aggp_ref, y4_ref, dinvp_ref, m_ref, b4_ref, out_ref):
    agg = aggp_ref[0] + aggp_ref[1]
    dinv = dinvp_ref[...]
    m = m_ref[...]
    h4 = dinv * (m * agg + y4_ref[...]) + b4_ref[...][None, :]
    s = lax.dot_general(m, h4, (((0,), (0,)), ((), ())),
                        preferred_element_type=F32)
    @pl.when(pl.program_id(0) == 0)
    def _():
        out_ref[...] = jnp.zeros_like(out_ref)
    out_ref[...] += s / F32(K)


def _build_tc(interpret=False):
    tca = pl.pallas_call(
        _tca_body,
        grid=(NB,),
        in_specs=[_fspec(), _wspec((F, F)), _pspec(1)],
        out_specs=(_fspec(), _vspec()),
        out_shape=(jax.ShapeDtypeStruct((N, F), F32),
                   jax.ShapeDtypeStruct((N, 1), F32)),
        interpret=interpret)
    tcb = pl.pallas_call(
        _tcb_body,
        out_shape=(jax.ShapeDtypeStruct((N, F), F32),
                   jax.ShapeDtypeStruct((2, N), F32)),
        interpret=interpret)
    tcc = pl.pallas_call(
        _tcc_body,
        grid=(NB,),
        in_specs=[_fspec(), _vspec(), _wspec((F, F))],
        out_specs=_fspec(),
        out_shape=jax.ShapeDtypeStruct((N, F), F32),
        interpret=interpret)
    tcp = pl.pallas_call(
        _tcp_body,
        out_shape=jax.ShapeDtypeStruct((E // 128, 128), jnp.int32),
        interpret=interpret)
    tcd = pl.pallas_call(
        _tcd_body,
        grid=(NB,),
        in_specs=[_pspec(1), _vspec(), _fspec()],
        out_specs=(_fspec(), _vspec()),
        out_shape=(jax.ShapeDtypeStruct((N, F), F32),
                   jax.ShapeDtypeStruct((N, 1), F32)),
        interpret=interpret)
    tce = pl.pallas_call(
        _tce_body,
        grid=(NB,),
        in_specs=[_pspec(F), _fspec(), _vspec(), _vspec(), _wspec((F,)),
                  _wspec((F, F))],
        out_specs=_fspec(),
        out_shape=jax.ShapeDtypeStruct((N, F), F32),
        interpret=interpret)
    tcg = pl.pallas_call(
        _tcg_body,
        grid=(NB,),
        in_specs=[_pspec(F), _fspec(), _vspec(), _vspec(), _wspec((F,))],
        out_specs=pl.BlockSpec((1, F), lambda i: (0, 0)),
        out_shape=jax.ShapeDtypeStruct((1, F), F32),
        interpret=interpret)
    return tca, tcb, tcc, tcd, tce, tcg, tcp


_tca, _tcb, _tcc, _tcd, _tce, _tcg, _tcp = _build_tc()


def kernel(x, edge_index, batch, W1, b1, p, W2, b2, W3, b3, W4, b4):
    del batch
    _sc_rows = _get_sc_scatter(F)
    _sc_deg = _get_sc_scatter(None)
    packed = _tcp(edge_index.reshape(2, E // 128, 128)).reshape(E)
    ones_n = jnp.ones((N,), F32)

    deg0p = _sc_deg(ones_n, packed)
    y1, dinv0 = _tca(x, W1, deg0p.reshape(2, N_PAD, 1))
    agg1p = _sc_rows(y1, packed)
    h, sm2 = _tcb(agg1p, y1, dinv0, b1, p)
    m = sm2[0].reshape(N, 1)
    xw2 = _tcc(h, sm2[1].reshape(N, 1), W2)
    degap = _sc_deg(sm2[0], packed)
    y2, dinvp = _tcd(degap.reshape(2, N_PAD, 1), m, xw2)
    agg2p = _sc_rows(y2, packed)
    y3 = _tce(agg2p, y2, dinvp, m, b2, W3)
    agg3p = _sc_rows(y3, packed)
    y4 = _tce(agg3p, y3, dinvp, m, b3, W4)
    agg4p = _sc_rows(y4, packed)
    return _tcg(agg4p, y4, dinvp, m, b4)
